# baseline (speedup 1.0000x reference)
import functools

import jax
import jax.numpy as jnp
from jax import lax
from jax.experimental import pallas as pl
from jax.experimental.pallas import tpu as pltpu
from jax.experimental.pallas import tpu_sc as plsc

_NB = 3


def _emb_kernel(S, D, V, B, NC, NW):
    BLK = B // NW
    mesh = plsc.VectorSubcoreMesh(core_axis_name="c", subcore_axis_name="s")
    n_iter = S // _NB
    tail = S - _NB * n_iter

    @functools.partial(
        pl.kernel,
        mesh=mesh,
        out_type=jax.ShapeDtypeStruct((S, D, B), jnp.float32),
        scratch_types=[
            pltpu.VMEM((V * D,), jnp.float32),
        ] + [pltpu.VMEM((BLK,), jnp.int32) for _ in range(_NB)]
          + [pltpu.VMEM((D, BLK), jnp.float32) for _ in range(_NB)]
          + [pltpu.SemaphoreType.DMA for _ in range(2 * _NB)],
        compiler_params=pltpu.CompilerParams(needs_layout_passes=False),
    )
    def k(wt_hbm, xt_hbm, out_hbm, wt_v, *rest):
        idxs = rest[:_NB]
        bufs = rest[_NB:2 * _NB]
        isems = rest[2 * _NB:3 * _NB]
        wsems = rest[3 * _NB:4 * _NB]
        wid = lax.axis_index("s") * NC + lax.axis_index("c")
        col0 = wid * BLK

        def compute(s, idx_v, buf, wsem):
            cvecs = [idx_v[pl.ds(g * 16, 16)] for g in range(BLK // 16)]
            blocks = [(g, d0) for g in range(BLK // 16)
                      for d0 in range(0, D, 16)]
            prev = None
            for g, d0 in blocks:
                cvec = cvecs[g]
                cur = []
                for u in range(16):
                    cur.append(
                        plsc.load_gather(wt_v, [cvec + (d0 + u) * V]))
                    if prev is not None:
                        pg, pd0, pvals = prev
                        buf[pd0 + u, pl.ds(pg * 16, 16)] = pvals[u]
                prev = (g, d0, cur)
            pg, pd0, pvals = prev
            for u in range(16):
                buf[pd0 + u, pl.ds(pg * 16, 16)] = pvals[u]
            pltpu.async_copy(buf, out_hbm.at[s, :, pl.ds(col0, BLK)], wsem)

        pltpu.sync_copy(wt_hbm, wt_v)
        for h in range(_NB):
            pltpu.async_copy(xt_hbm.at[h, pl.ds(col0, BLK)], idxs[h],
                             isems[h])

        def body(i, carry):
            for h in range(_NB):
                s = _NB * i + h
                idx_v, buf = idxs[h], bufs[h]
                pltpu.make_async_copy(
                    xt_hbm.at[s, pl.ds(col0, BLK)], idx_v, isems[h]).wait()

                def prefetch():
                    pltpu.async_copy(
                        xt_hbm.at[s + _NB, pl.ds(col0, BLK)], idx_v,
                        isems[h])

                if _NB * (n_iter - 1) + h + _NB < S:
                    prefetch()
                else:
                    pl.when(i < n_iter - 1)(prefetch)

                @pl.when(i > 0)
                def _():
                    pltpu.make_async_copy(
                        buf, out_hbm.at[s, :, pl.ds(col0, BLK)],
                        wsems[h]).wait()

                compute(s, idx_v, buf, wsems[h])
            return carry

        lax.fori_loop(0, n_iter, body, 0)
        for h in range(tail):
            s = _NB * n_iter + h
            pltpu.make_async_copy(
                xt_hbm.at[s, pl.ds(col0, BLK)], idxs[h], isems[h]).wait()
            pltpu.make_async_copy(
                bufs[h], out_hbm.at[s, :, pl.ds(col0, BLK)],
                wsems[h]).wait()
            compute(s, idxs[h], bufs[h], wsems[h])
        for h in range(_NB):
            pltpu.make_async_copy(
                bufs[h], out_hbm.at[0, :, pl.ds(col0, BLK)],
                wsems[h]).wait()

    return k


def kernel(x, weights):
    Bdim, S = x.shape
    V, D = weights.shape
    info = plsc.get_sparse_core_info()
    NC, NS = info.num_cores, info.num_subcores
    NW = NC * NS
    wt_flat = weights.astype(jnp.float32).T.reshape(V * D)
    xt = x.astype(jnp.int32).T
    k = _emb_kernel(S, D, V, Bdim, NC, NW)
    out_t = k(wt_flat, xt)
    return jnp.transpose(out_t, (2, 0, 1))


# speedup vs baseline: 1.2564x; 1.2564x over previous
import functools

import jax
import jax.numpy as jnp
from jax import lax
from jax.experimental import pallas as pl
from jax.experimental.pallas import tpu as pltpu
from jax.experimental.pallas import tpu_sc as plsc


def _emb_kernel(S, D, V, B, NC, NW):
    BLK = B // NW
    mesh = plsc.VectorSubcoreMesh(core_axis_name="c", subcore_axis_name="s")

    @functools.partial(
        pl.kernel,
        mesh=mesh,
        out_type=jax.ShapeDtypeStruct((S, D, B), jnp.float32),
        scratch_types=[
            pltpu.VMEM((V * D,), jnp.float32),
            pltpu.VMEM((BLK,), jnp.int32),
            pltpu.VMEM((BLK,), jnp.int32),
            pltpu.VMEM((D, BLK), jnp.float32),
            pltpu.VMEM((D, BLK), jnp.float32),
            pltpu.SemaphoreType.DMA,
            pltpu.SemaphoreType.DMA,
            pltpu.SemaphoreType.DMA,
            pltpu.SemaphoreType.DMA,
        ],
        compiler_params=pltpu.CompilerParams(needs_layout_passes=False),
    )
    def k(wt_hbm, xt_hbm, out_hbm, wt_v, idx0, idx1, buf0, buf1,
          isem0, isem1, wsem0, wsem1):
        wid = lax.axis_index("s") * NC + lax.axis_index("c")
        col0 = wid * BLK
        idxs = (idx0, idx1)
        bufs = (buf0, buf1)
        isems = (isem0, isem1)
        wsems = (wsem0, wsem1)

        pltpu.sync_copy(wt_hbm, wt_v)
        for h in range(2):
            pltpu.async_copy(xt_hbm.at[h, pl.ds(col0, BLK)], idxs[h],
                             isems[h])

        def body(i, carry):
            for h in range(2):
                s = 2 * i + h
                idx_v, buf = idxs[h], bufs[h]
                pltpu.make_async_copy(
                    xt_hbm.at[s, pl.ds(col0, BLK)], idx_v, isems[h]).wait()
                cvecs = [idx_v[pl.ds(g * 16, 16)] for g in range(BLK // 16)]

                @pl.when(i < (S // 2) - 1)
                def _():
                    pltpu.async_copy(
                        xt_hbm.at[s + 2, pl.ds(col0, BLK)], idx_v, isems[h])

                @pl.when(i > 0)
                def _():
                    pltpu.make_async_copy(
                        buf, out_hbm.at[s, :, pl.ds(col0, BLK)],
                        wsems[h]).wait()

                blocks = [(g, d0) for g in range(BLK // 16)
                          for d0 in range(0, D, 16)]
                prev = None
                for g, d0 in blocks:
                    cvec = cvecs[g]
                    cur = []
                    for u in range(16):
                        cur.append(
                            plsc.load_gather(wt_v, [cvec + (d0 + u) * V]))
                        if prev is not None:
                            pg, pd0, pvals = prev
                            buf[pd0 + u, pl.ds(pg * 16, 16)] = pvals[u]
                    prev = (g, d0, cur)
                pg, pd0, pvals = prev
                for u in range(16):
                    buf[pd0 + u, pl.ds(pg * 16, 16)] = pvals[u]
                pltpu.async_copy(
                    buf, out_hbm.at[s, :, pl.ds(col0, BLK)], wsems[h])
            return carry

        lax.fori_loop(0, S // 2, body, 0)
        for h in range(2):
            s = S - 2 + h
            pltpu.make_async_copy(
                bufs[h], out_hbm.at[s, :, pl.ds(col0, BLK)], wsems[h]).wait()

    return k


def kernel(x, weights):
    Bdim, S = x.shape
    V, D = weights.shape
    info = plsc.get_sparse_core_info()
    NC, NS = info.num_cores, info.num_subcores
    NW = NC * NS
    wt_flat = weights.astype(jnp.float32).T.reshape(V * D)
    xt = x.astype(jnp.int32).T
    k = _emb_kernel(S, D, V, Bdim, NC, NW)
    out_t = k(wt_flat, xt)
    return jnp.transpose(out_t, (2, 0, 1))


# speedup vs baseline: 1.2697x; 1.0106x over previous
import functools

import jax
import jax.numpy as jnp
from jax import lax
from jax.experimental import pallas as pl
from jax.experimental.pallas import tpu as pltpu
from jax.experimental.pallas import tpu_sc as plsc


def _emb_kernel(S, D, V, B, NC, NW):
    BLK = B // NW
    mesh = plsc.VectorSubcoreMesh(core_axis_name="c", subcore_axis_name="s")

    @functools.partial(
        pl.kernel,
        mesh=mesh,
        out_type=jax.ShapeDtypeStruct((S, D, B), jnp.float32),
        scratch_types=[
            pltpu.VMEM((V * D,), jnp.float32),
            pltpu.VMEM((BLK,), jnp.int32),
            pltpu.VMEM((BLK,), jnp.int32),
            pltpu.VMEM((D, BLK), jnp.float32),
            pltpu.VMEM((D, BLK), jnp.float32),
            pltpu.SemaphoreType.DMA,
            pltpu.SemaphoreType.DMA,
            pltpu.SemaphoreType.DMA,
            pltpu.SemaphoreType.DMA,
        ],
        compiler_params=pltpu.CompilerParams(needs_layout_passes=False),
    )
    def k(wt_hbm, xt_hbm, out_hbm, wt_v, idx0, idx1, buf0, buf1,
          isem0, isem1, wsem0, wsem1):
        wid = lax.axis_index("s") * NC + lax.axis_index("c")
        col0 = wid * BLK
        idxs = (idx0, idx1)
        bufs = (buf0, buf1)
        isems = (isem0, isem1)
        wsems = (wsem0, wsem1)

        pltpu.sync_copy(wt_hbm, wt_v)
        for h in range(2):
            pltpu.async_copy(xt_hbm.at[h, pl.ds(col0, BLK)], idxs[h],
                             isems[h])

        def body(i, carry):
            for h in range(2):
                s = 2 * i + h
                idx_v, buf = idxs[h], bufs[h]
                pltpu.make_async_copy(
                    xt_hbm.at[s, pl.ds(col0, BLK)], idx_v, isems[h]).wait()
                cvecs = [idx_v[pl.ds(g * 16, 16)] for g in range(BLK // 16)]

                @pl.when(i < (S // 2) - 1)
                def _():
                    pltpu.async_copy(
                        xt_hbm.at[s + 2, pl.ds(col0, BLK)], idx_v, isems[h])

                @pl.when(i > 0)
                def _():
                    pltpu.make_async_copy(
                        buf.at[pl.ds(0, 32)],
                        out_hbm.at[s, pl.ds(0, 32), pl.ds(col0, BLK)],
                        wsems[h]).wait()
                    pltpu.make_async_copy(
                        buf.at[pl.ds(32, 32)],
                        out_hbm.at[s, pl.ds(32, 32), pl.ds(col0, BLK)],
                        wsems[h]).wait()

                for half in range(2):
                    blocks = [(g, d0) for g in range(BLK // 16)
                              for d0 in range(half * 32, half * 32 + 32, 16)]
                    prev = None
                    for g, d0 in blocks:
                        cvec = cvecs[g]
                        cur = []
                        for u in range(16):
                            cur.append(
                                plsc.load_gather(wt_v, [cvec + (d0 + u) * V]))
                            if prev is not None:
                                pg, pd0, pvals = prev
                                buf[pd0 + u, pl.ds(pg * 16, 16)] = pvals[u]
                        prev = (g, d0, cur)
                    pg, pd0, pvals = prev
                    for u in range(16):
                        buf[pd0 + u, pl.ds(pg * 16, 16)] = pvals[u]
                    pltpu.async_copy(
                        buf.at[pl.ds(half * 32, 32)],
                        out_hbm.at[s, pl.ds(half * 32, 32), pl.ds(col0, BLK)],
                        wsems[h])
            return carry

        lax.fori_loop(0, S // 2, body, 0)
        for h in range(2):
            s = S - 2 + h
            pltpu.make_async_copy(
                bufs[h].at[pl.ds(0, 32)],
                out_hbm.at[s, pl.ds(0, 32), pl.ds(col0, BLK)],
                wsems[h]).wait()
            pltpu.make_async_copy(
                bufs[h].at[pl.ds(32, 32)],
                out_hbm.at[s, pl.ds(32, 32), pl.ds(col0, BLK)],
                wsems[h]).wait()

    return k


def kernel(x, weights):
    Bdim, S = x.shape
    V, D = weights.shape
    info = plsc.get_sparse_core_info()
    NC, NS = info.num_cores, info.num_subcores
    NW = NC * NS
    wt_flat = weights.astype(jnp.float32).T.reshape(V * D)
    xt = x.astype(jnp.int32).T
    k = _emb_kernel(S, D, V, Bdim, NC, NW)
    out_t = k(wt_flat, xt)
    return jnp.transpose(out_t, (2, 0, 1))


# speedup vs baseline: 1.3306x; 1.0480x over previous
import functools

import jax
import jax.numpy as jnp
from jax import lax
from jax.experimental import pallas as pl
from jax.experimental.pallas import tpu as pltpu
from jax.experimental.pallas import tpu_sc as plsc


def _emb_kernel(S, D, V, B, NC, NW):
    BLK = B // NW
    mesh = plsc.VectorSubcoreMesh(core_axis_name="c", subcore_axis_name="s")

    @functools.partial(
        pl.kernel,
        mesh=mesh,
        out_type=jax.ShapeDtypeStruct((S, D, B), jnp.float32),
        scratch_types=[
            pltpu.VMEM((V * D,), jnp.float32),
            pltpu.VMEM((BLK,), jnp.int32),
            pltpu.VMEM((BLK,), jnp.int32),
            pltpu.VMEM((D, BLK), jnp.float32),
            pltpu.VMEM((D, BLK), jnp.float32),
            pltpu.SemaphoreType.DMA,
            pltpu.SemaphoreType.DMA,
            pltpu.SemaphoreType.DMA,
            pltpu.SemaphoreType.DMA,
        ],
        compiler_params=pltpu.CompilerParams(needs_layout_passes=False),
    )
    def k(wt_hbm, xt_hbm, out_hbm, wt_v, idx0, idx1, buf0, buf1,
          isem0, isem1, wsem0, wsem1):
        wid = lax.axis_index("s") * NC + lax.axis_index("c")
        col0 = wid * BLK
        idxs = (idx0, idx1)
        bufs = (buf0, buf1)
        isems = (isem0, isem1)
        wsems = (wsem0, wsem1)

        pltpu.sync_copy(wt_hbm, wt_v)
        for h in range(2):
            pltpu.async_copy(xt_hbm.at[h, pl.ds(col0, BLK)], idxs[h],
                             isems[h])

        def body(i, carry):
            for h in range(2):
                s = 2 * i + h
                idx_v, buf = idxs[h], bufs[h]
                pltpu.make_async_copy(
                    xt_hbm.at[s, pl.ds(col0, BLK)], idx_v, isems[h]).wait()
                cvecs = [idx_v[pl.ds(g * 16, 16)] for g in range(BLK // 16)]

                @pl.when(i < (S // 2) - 1)
                def _():
                    pltpu.async_copy(
                        xt_hbm.at[s + 2, pl.ds(col0, BLK)], idx_v, isems[h])

                @pl.when(i > 0)
                def _():
                    pltpu.make_async_copy(
                        buf, out_hbm.at[s, :, pl.ds(col0, BLK)],
                        wsems[h]).wait()

                for half in range(4):
                    blocks = [(g, half * 16) for g in range(BLK // 16)]
                    prev = None
                    for g, d0 in blocks:
                        cvec = cvecs[g]
                        cur = []
                        for u in range(16):
                            cur.append(
                                plsc.load_gather(wt_v, [cvec + (d0 + u) * V]))
                            if prev is not None:
                                pg, pd0, pvals = prev
                                buf[pd0 + u, pl.ds(pg * 16, 16)] = pvals[u]
                        prev = (g, d0, cur)
                    pg, pd0, pvals = prev
                    for u in range(16):
                        buf[pd0 + u, pl.ds(pg * 16, 16)] = pvals[u]
                    pltpu.async_copy(
                        buf.at[pl.ds(half * 16, 16)],
                        out_hbm.at[s, pl.ds(half * 16, 16), pl.ds(col0, BLK)],
                        wsems[h])
            return carry

        lax.fori_loop(0, S // 2, body, 0)
        for h in range(2):
            s = S - 2 + h
            pltpu.make_async_copy(
                bufs[h], out_hbm.at[s, :, pl.ds(col0, BLK)], wsems[h]).wait()

    return k


def kernel(x, weights):
    Bdim, S = x.shape
    V, D = weights.shape
    info = plsc.get_sparse_core_info()
    NC, NS = info.num_cores, info.num_subcores
    NW = NC * NS
    wt_flat = weights.astype(jnp.float32).T.reshape(V * D)
    xt = x.astype(jnp.int32).T
    k = _emb_kernel(S, D, V, Bdim, NC, NW)
    out_t = k(wt_flat, xt)
    return jnp.transpose(out_t, (2, 0, 1))


# speedup vs baseline: 1.3626x; 1.0240x over previous
import functools

import jax
import jax.numpy as jnp
from jax import lax
from jax.experimental import pallas as pl
from jax.experimental.pallas import tpu as pltpu
from jax.experimental.pallas import tpu_sc as plsc


def _emb_kernel(S, D, V, B, NC, NW):
    BLK = B // NW
    mesh = plsc.VectorSubcoreMesh(core_axis_name="c", subcore_axis_name="s")

    @functools.partial(
        pl.kernel,
        mesh=mesh,
        out_type=jax.ShapeDtypeStruct((S, D, B), jnp.float32),
        scratch_types=[
            pltpu.VMEM((V * D,), jnp.float32),
            pltpu.VMEM((BLK,), jnp.int32),
            pltpu.VMEM((BLK,), jnp.int32),
            pltpu.VMEM((D, BLK), jnp.float32),
            pltpu.VMEM((D, BLK), jnp.float32),
            pltpu.SemaphoreType.DMA,
            pltpu.SemaphoreType.DMA,
            pltpu.SemaphoreType.DMA,
            pltpu.SemaphoreType.DMA,
        ],
        compiler_params=pltpu.CompilerParams(needs_layout_passes=False),
    )
    def k(wt_hbm, xt_hbm, out_hbm, wt_v, idx0, idx1, buf0, buf1,
          isem0, isem1, wsem0, wsem1):
        wid = lax.axis_index("s") * NC + lax.axis_index("c")
        col0 = wid * BLK
        idxs = (idx0, idx1)
        bufs = (buf0, buf1)
        isems = (isem0, isem1)
        wsems = (wsem0, wsem1)

        pltpu.sync_copy(wt_hbm, wt_v)
        for h in range(2):
            pltpu.async_copy(xt_hbm.at[h, pl.ds(col0, BLK)], idxs[h],
                             isems[h])

        def body(i, carry):
            for h in range(2):
                s = 2 * i + h
                idx_v, buf = idxs[h], bufs[h]
                pltpu.make_async_copy(
                    xt_hbm.at[s, pl.ds(col0, BLK)], idx_v, isems[h]).wait()
                cvecs = [idx_v[pl.ds(g * 16, 16)] for g in range(BLK // 16)]

                @pl.when(i < (S // 2) - 1)
                def _():
                    pltpu.async_copy(
                        xt_hbm.at[s + 2, pl.ds(col0, BLK)], idx_v, isems[h])

                @pl.when(i > 0)
                def _():
                    pltpu.make_async_copy(
                        buf, out_hbm.at[s, :, pl.ds(col0, BLK)],
                        wsems[h]).wait()

                blocks = [(g, q * 16) for q in range(4)
                          for g in range(BLK // 16)]
                prev = None
                for bi, (g, d0) in enumerate(blocks):
                    cvec = cvecs[g]
                    cur = []
                    for u in range(16):
                        cur.append(
                            plsc.load_gather(wt_v, [cvec + (d0 + u) * V]))
                        if prev is not None:
                            pg, pd0, pvals = prev
                            buf[pd0 + u, pl.ds(pg * 16, 16)] = pvals[u]
                    prev = (g, d0, cur)
                    if bi % 8 == 0 and bi > 0:
                        q = bi // 8 - 1
                        pltpu.async_copy(
                            buf.at[pl.ds(q * 16, 16)],
                            out_hbm.at[s, pl.ds(q * 16, 16),
                                       pl.ds(col0, BLK)],
                            wsems[h])
                pg, pd0, pvals = prev
                for u in range(16):
                    buf[pd0 + u, pl.ds(pg * 16, 16)] = pvals[u]
                pltpu.async_copy(
                    buf.at[pl.ds(48, 16)],
                    out_hbm.at[s, pl.ds(48, 16), pl.ds(col0, BLK)],
                    wsems[h])
            return carry

        lax.fori_loop(0, S // 2, body, 0)
        for h in range(2):
            s = S - 2 + h
            pltpu.make_async_copy(
                bufs[h], out_hbm.at[s, :, pl.ds(col0, BLK)], wsems[h]).wait()

    return k


def kernel(x, weights):
    Bdim, S = x.shape
    V, D = weights.shape
    info = plsc.get_sparse_core_info()
    NC, NS = info.num_cores, info.num_subcores
    NW = NC * NS
    wt_flat = weights.astype(jnp.float32).T.reshape(V * D)
    xt = x.astype(jnp.int32).T
    k = _emb_kernel(S, D, V, Bdim, NC, NW)
    out_t = k(wt_flat, xt)
    return jnp.transpose(out_t, (2, 0, 1))
